# baseline (speedup 1.0000x reference)
import functools

import jax
import jax.numpy as jnp
import numpy as np
from jax.experimental import pallas as pl
from jax.experimental.pallas import tpu as pltpu

_B, _N, _F_IN, _HID, _ITERS = 128, 200, 8, 256, 3
_SCALE = 1.0 / np.sqrt(_HID)


def _mm(a, b):
    return jax.lax.dot_general(
        a, b, (((1,), (0,)), ((), ())), preferred_element_type=jnp.float32
    )


def _mpnn_kernel(jets_ref, w_emb_ref, b_emb_ref,
                 w_mp0_ref, b_mp0_ref, w_mp1_ref, b_mp1_ref,
                 w_mp2_ref, b_mp2_ref,
                 w_r1_ref, b_r1_ref, w_r2_ref, b_r2_ref,
                 out_ref, a_ref):
    x = jets_ref[0]
    h = jnp.tanh(_mm(x, w_emb_ref[...]) + b_emb_ref[...])
    hb = h.astype(jnp.bfloat16)

    a = None
    for w_ref, b_ref in ((w_mp0_ref, b_mp0_ref),
                         (w_mp1_ref, b_mp1_ref),
                         (w_mp2_ref, b_mp2_ref)):
        logits = jax.lax.dot_general(
            hb, hb, (((1,), (1,)), ((), ())),
            preferred_element_type=jnp.float32) * _SCALE
        m = jnp.max(logits, axis=-1, keepdims=True)
        p = jnp.exp(logits - m)
        a = p / jnp.sum(p, axis=-1, keepdims=True)
        msg = _mm(a.astype(jnp.bfloat16), hb)
        w = w_ref[...]
        upd = _mm(hb, w[:_HID]) + _mm(msg.astype(jnp.bfloat16), w[_HID:]) \
            + b_ref[...]
        hb = jnp.tanh(upd).astype(jnp.bfloat16)

    r = jnp.tanh(_mm(hb, w_r1_ref[...]) + b_r1_ref[...])
    r2 = _mm(r.astype(jnp.bfloat16), w_r2_ref[...])
    out_ref[0] = jnp.sum(r2, axis=0, keepdims=True) + _N * b_r2_ref[...]
    a_ref[0] = a


def kernel(jets, W_emb, b_emb, W_mp0, b_mp0, W_mp1, b_mp1, W_mp2, b_mp2,
           W_r1, b_r1, W_r2, b_r2):
    B, N, F_IN = jets.shape
    HID = W_emb.shape[1]

    def rep(shape):
        return pl.BlockSpec(shape, lambda b: (0,) * len(shape))

    jets = jets.astype(jnp.bfloat16)
    W_emb = W_emb.astype(jnp.bfloat16)
    W_mp0 = W_mp0.astype(jnp.bfloat16)
    W_mp1 = W_mp1.astype(jnp.bfloat16)
    W_mp2 = W_mp2.astype(jnp.bfloat16)
    W_r1 = W_r1.astype(jnp.bfloat16)
    W_r2 = W_r2.astype(jnp.bfloat16)
    b_emb2 = b_emb.reshape(1, HID)
    b_mp0_2 = b_mp0.reshape(1, HID)
    b_mp1_2 = b_mp1.reshape(1, HID)
    b_mp2_2 = b_mp2.reshape(1, HID)
    b_r1_2 = b_r1.reshape(1, HID)
    b_r2_2 = b_r2.reshape(1, HID)

    out, a = pl.pallas_call(
        _mpnn_kernel,
        grid=(B,),
        in_specs=[
            pl.BlockSpec((1, N, F_IN), lambda b: (b, 0, 0)),
            rep((F_IN, HID)), rep((1, HID)),
            rep((2 * HID, HID)), rep((1, HID)),
            rep((2 * HID, HID)), rep((1, HID)),
            rep((2 * HID, HID)), rep((1, HID)),
            rep((HID, HID)), rep((1, HID)),
            rep((HID, HID)), rep((1, HID)),
        ],
        out_specs=[
            pl.BlockSpec((1, 1, HID), lambda b: (b, 0, 0)),
            pl.BlockSpec((1, N, N), lambda b: (b, 0, 0)),
        ],
        out_shape=[
            jax.ShapeDtypeStruct((B, 1, HID), jnp.float32),
            jax.ShapeDtypeStruct((B, N, N), jnp.float32),
        ],
        compiler_params=pltpu.CompilerParams(
            dimension_semantics=("parallel",),
        ),
    )(jets, W_emb, b_emb2, W_mp0, b_mp0_2, W_mp1, b_mp1_2, W_mp2, b_mp2_2,
      W_r1, b_r1_2, W_r2, b_r2_2)
    return (out.reshape(B, HID), a)


# speedup vs baseline: 1.1233x; 1.1233x over previous
import jax
import jax.numpy as jnp
import numpy as np
from jax.experimental import pallas as pl
from jax.experimental.pallas import tpu as pltpu

_B, _N, _F_IN, _HID, _ITERS = 128, 200, 8, 256, 3
_SCALE = 1.0 / np.sqrt(_HID)
_J = 2


def _mm(a, b):
    return jax.lax.dot_general(
        a, b, (((1,), (0,)), ((), ())), preferred_element_type=jnp.float32
    )


def _jet_chain(x, w_emb, b_emb, mp_params, w_r1, b_r1, w_r2, b_r2):
    h = jnp.tanh(_mm(x, w_emb) + b_emb)
    a = None
    for w, b in mp_params:
        logits = jax.lax.dot_general(
            h, h, (((1,), (1,)), ((), ())),
            preferred_element_type=jnp.float32) * _SCALE
        m = jnp.max(logits, axis=-1, keepdims=True)
        p = jnp.exp(logits - m)
        a = p / jnp.sum(p, axis=-1, keepdims=True)
        msg = _mm(a, h)
        h = jnp.tanh(_mm(h, w[:_HID]) + _mm(msg, w[_HID:]) + b)
    r = jnp.tanh(_mm(h, w_r1) + b_r1)
    r2 = _mm(r, w_r2)
    out = jnp.sum(r2, axis=0, keepdims=True) + _N * b_r2
    return out, a


def _mpnn_kernel(jets_ref, w_emb_ref, b_emb_ref,
                 w_mp0_ref, b_mp0_ref, w_mp1_ref, b_mp1_ref,
                 w_mp2_ref, b_mp2_ref,
                 w_r1_ref, b_r1_ref, w_r2_ref, b_r2_ref,
                 out_ref, a_ref):
    w_emb = w_emb_ref[...]
    b_emb = b_emb_ref[...]
    mp_params = [(w_mp0_ref[...], b_mp0_ref[...]),
                 (w_mp1_ref[...], b_mp1_ref[...]),
                 (w_mp2_ref[...], b_mp2_ref[...])]
    w_r1 = w_r1_ref[...]
    b_r1 = b_r1_ref[...]
    w_r2 = w_r2_ref[...]
    b_r2 = b_r2_ref[...]
    for j in range(_J):
        out, a = _jet_chain(jets_ref[j], w_emb, b_emb, mp_params,
                            w_r1, b_r1, w_r2, b_r2)
        out_ref[j] = out
        a_ref[j] = a


def kernel(jets, W_emb, b_emb, W_mp0, b_mp0, W_mp1, b_mp1, W_mp2, b_mp2,
           W_r1, b_r1, W_r2, b_r2):
    B, N, F_IN = jets.shape
    HID = W_emb.shape[1]

    def rep(shape):
        return pl.BlockSpec(shape, lambda b: (0,) * len(shape))

    b_emb2 = b_emb.reshape(1, HID)
    b_mp0_2 = b_mp0.reshape(1, HID)
    b_mp1_2 = b_mp1.reshape(1, HID)
    b_mp2_2 = b_mp2.reshape(1, HID)
    b_r1_2 = b_r1.reshape(1, HID)
    b_r2_2 = b_r2.reshape(1, HID)

    out, a = pl.pallas_call(
        _mpnn_kernel,
        grid=(B // _J,),
        in_specs=[
            pl.BlockSpec((_J, N, F_IN), lambda b: (b, 0, 0)),
            rep((F_IN, HID)), rep((1, HID)),
            rep((2 * HID, HID)), rep((1, HID)),
            rep((2 * HID, HID)), rep((1, HID)),
            rep((2 * HID, HID)), rep((1, HID)),
            rep((HID, HID)), rep((1, HID)),
            rep((HID, HID)), rep((1, HID)),
        ],
        out_specs=[
            pl.BlockSpec((_J, 1, HID), lambda b: (b, 0, 0)),
            pl.BlockSpec((_J, N, N), lambda b: (b, 0, 0)),
        ],
        out_shape=[
            jax.ShapeDtypeStruct((B, 1, HID), jnp.float32),
            jax.ShapeDtypeStruct((B, N, N), jnp.float32),
        ],
        compiler_params=pltpu.CompilerParams(
            dimension_semantics=("parallel",),
        ),
    )(jets, W_emb, b_emb2, W_mp0, b_mp0_2, W_mp1, b_mp1_2, W_mp2, b_mp2_2,
      W_r1, b_r1_2, W_r2, b_r2_2)
    return (out.reshape(B, HID), a)


# speedup vs baseline: 2.6899x; 2.3946x over previous
import jax
import jax.numpy as jnp
import numpy as np
from jax.experimental import pallas as pl
from jax.experimental.pallas import tpu as pltpu

_B, _N, _F_IN, _HID, _ITERS = 128, 200, 8, 256, 3
_SCALE = 1.0 / np.sqrt(_HID)
_J = 4


def _mm(a, b):
    return jax.lax.dot_general(
        a, b, (((1,), (0,)), ((), ())), preferred_element_type=jnp.float32
    )


def _mpnn_kernel(jets_ref, w_emb_ref, b_emb_ref,
                 w_mp0_ref, b_mp0_ref, w_mp1_ref, b_mp1_ref,
                 w_mp2_ref, b_mp2_ref,
                 w_r1_ref, b_r1_ref, w_r2_ref, b_r2_ref,
                 out_ref, a_ref):
    w_emb = w_emb_ref[...]
    b_emb = b_emb_ref[...]
    mp_params = [(w_mp0_ref[...], b_mp0_ref[...]),
                 (w_mp1_ref[...], b_mp1_ref[...]),
                 (w_mp2_ref[...], b_mp2_ref[...])]
    w_r1 = w_r1_ref[...]
    b_r1 = b_r1_ref[...]
    w_r2 = w_r2_ref[...]
    b_r2 = b_r2_ref[...]

    hs = [jnp.tanh(_mm(jets_ref[j], w_emb) + b_emb) for j in range(_J)]
    a_s = [None] * _J
    for w, b in mp_params:
        logits = [jax.lax.dot_general(
            h, h, (((1,), (1,)), ((), ())),
            preferred_element_type=jnp.float32) * _SCALE for h in hs]
        ms = [jnp.max(l, axis=-1, keepdims=True) for l in logits]
        ps = [jnp.exp(l - m) for l, m in zip(logits, ms)]
        a_s = [p / jnp.sum(p, axis=-1, keepdims=True) for p in ps]
        msgs = [_mm(a, h) for a, h in zip(a_s, hs)]
        hs = [jnp.tanh(_mm(h, w[:_HID]) + _mm(msg, w[_HID:]) + b)
              for h, msg in zip(hs, msgs)]
    rs = [jnp.tanh(_mm(h, w_r1) + b_r1) for h in hs]
    r2s = [_mm(r, w_r2) for r in rs]
    for j in range(_J):
        out_ref[j] = jnp.sum(r2s[j], axis=0, keepdims=True) + _N * b_r2
        a_ref[j] = a_s[j]


def kernel(jets, W_emb, b_emb, W_mp0, b_mp0, W_mp1, b_mp1, W_mp2, b_mp2,
           W_r1, b_r1, W_r2, b_r2):
    B, N, F_IN = jets.shape
    HID = W_emb.shape[1]

    def rep(shape):
        return pl.BlockSpec(shape, lambda b: (0,) * len(shape))

    b_emb2 = b_emb.reshape(1, HID)
    b_mp0_2 = b_mp0.reshape(1, HID)
    b_mp1_2 = b_mp1.reshape(1, HID)
    b_mp2_2 = b_mp2.reshape(1, HID)
    b_r1_2 = b_r1.reshape(1, HID)
    b_r2_2 = b_r2.reshape(1, HID)

    out, a = pl.pallas_call(
        _mpnn_kernel,
        grid=(B // _J,),
        in_specs=[
            pl.BlockSpec((_J, N, F_IN), lambda b: (b, 0, 0)),
            rep((F_IN, HID)), rep((1, HID)),
            rep((2 * HID, HID)), rep((1, HID)),
            rep((2 * HID, HID)), rep((1, HID)),
            rep((2 * HID, HID)), rep((1, HID)),
            rep((HID, HID)), rep((1, HID)),
            rep((HID, HID)), rep((1, HID)),
        ],
        out_specs=[
            pl.BlockSpec((_J, 1, HID), lambda b: (b, 0, 0)),
            pl.BlockSpec((_J, N, N), lambda b: (b, 0, 0)),
        ],
        out_shape=[
            jax.ShapeDtypeStruct((B, 1, HID), jnp.float32),
            jax.ShapeDtypeStruct((B, N, N), jnp.float32),
        ],
        compiler_params=pltpu.CompilerParams(
            dimension_semantics=("parallel",),
        ),
    )(jets, W_emb, b_emb2, W_mp0, b_mp0_2, W_mp1, b_mp1_2, W_mp2, b_mp2_2,
      W_r1, b_r1_2, W_r2, b_r2_2)
    return (out.reshape(B, HID), a)


# speedup vs baseline: 3.3670x; 1.2517x over previous
import jax
import jax.numpy as jnp
import numpy as np
from jax.experimental import pallas as pl
from jax.experimental.pallas import tpu as pltpu

_B, _N, _F_IN, _HID, _ITERS = 128, 200, 8, 256, 3
_SCALE = 1.0 / np.sqrt(_HID)
_J = 8


def _mm(a, b):
    return jax.lax.dot_general(
        a, b, (((1,), (0,)), ((), ())), preferred_element_type=jnp.float32
    )


def _mpnn_kernel(jets_ref, w_emb_ref, b_emb_ref,
                 w_mp0_ref, b_mp0_ref, w_mp1_ref, b_mp1_ref,
                 w_mp2_ref, b_mp2_ref,
                 w_r1_ref, b_r1_ref, w_r2_ref, b_r2_ref,
                 out_ref, a_ref):
    w_emb = w_emb_ref[...]
    b_emb = b_emb_ref[...]
    mp_params = [(w_mp0_ref[...], b_mp0_ref[...]),
                 (w_mp1_ref[...], b_mp1_ref[...]),
                 (w_mp2_ref[...], b_mp2_ref[...])]
    w_r1 = w_r1_ref[...]
    b_r1 = b_r1_ref[...]
    w_r2 = w_r2_ref[...]
    b_r2 = b_r2_ref[...]

    hs = [jnp.tanh(_mm(jets_ref[j], w_emb) + b_emb) for j in range(_J)]
    a_s = [None] * _J
    for w, b in mp_params:
        logits = [jax.lax.dot_general(
            h, h, (((1,), (1,)), ((), ())),
            preferred_element_type=jnp.float32) * _SCALE for h in hs]
        ms = [jnp.max(l, axis=-1, keepdims=True) for l in logits]
        ps = [jnp.exp(l - m) for l, m in zip(logits, ms)]
        a_s = [p / jnp.sum(p, axis=-1, keepdims=True) for p in ps]
        msgs = [_mm(a, h) for a, h in zip(a_s, hs)]
        hs = [jnp.tanh(_mm(h, w[:_HID]) + _mm(msg, w[_HID:]) + b)
              for h, msg in zip(hs, msgs)]
    rs = [jnp.tanh(_mm(h, w_r1) + b_r1) for h in hs]
    r2s = [_mm(r, w_r2) for r in rs]
    for j in range(_J):
        out_ref[j] = jnp.sum(r2s[j], axis=0, keepdims=True) + _N * b_r2
        a_ref[j] = a_s[j]


def kernel(jets, W_emb, b_emb, W_mp0, b_mp0, W_mp1, b_mp1, W_mp2, b_mp2,
           W_r1, b_r1, W_r2, b_r2):
    B, N, F_IN = jets.shape
    HID = W_emb.shape[1]

    def rep(shape):
        return pl.BlockSpec(shape, lambda b: (0,) * len(shape))

    b_emb2 = b_emb.reshape(1, HID)
    b_mp0_2 = b_mp0.reshape(1, HID)
    b_mp1_2 = b_mp1.reshape(1, HID)
    b_mp2_2 = b_mp2.reshape(1, HID)
    b_r1_2 = b_r1.reshape(1, HID)
    b_r2_2 = b_r2.reshape(1, HID)

    out, a = pl.pallas_call(
        _mpnn_kernel,
        grid=(B // _J,),
        in_specs=[
            pl.BlockSpec((_J, N, F_IN), lambda b: (b, 0, 0)),
            rep((F_IN, HID)), rep((1, HID)),
            rep((2 * HID, HID)), rep((1, HID)),
            rep((2 * HID, HID)), rep((1, HID)),
            rep((2 * HID, HID)), rep((1, HID)),
            rep((HID, HID)), rep((1, HID)),
            rep((HID, HID)), rep((1, HID)),
        ],
        out_specs=[
            pl.BlockSpec((_J, 1, HID), lambda b: (b, 0, 0)),
            pl.BlockSpec((_J, N, N), lambda b: (b, 0, 0)),
        ],
        out_shape=[
            jax.ShapeDtypeStruct((B, 1, HID), jnp.float32),
            jax.ShapeDtypeStruct((B, N, N), jnp.float32),
        ],
        compiler_params=pltpu.CompilerParams(
            dimension_semantics=("parallel",),
        ),
    )(jets, W_emb, b_emb2, W_mp0, b_mp0_2, W_mp1, b_mp1_2, W_mp2, b_mp2_2,
      W_r1, b_r1_2, W_r2, b_r2_2)
    return (out.reshape(B, HID), a)


# speedup vs baseline: 3.3957x; 1.0085x over previous
import jax
import jax.numpy as jnp
import numpy as np
from jax.experimental import pallas as pl
from jax.experimental.pallas import tpu as pltpu

_B, _N, _F_IN, _HID, _ITERS = 128, 200, 8, 256, 3
_SCALE = 1.0 / np.sqrt(_HID)
_J = 16


def _mm(a, b):
    return jax.lax.dot_general(
        a, b, (((1,), (0,)), ((), ())), preferred_element_type=jnp.float32
    )


def _mpnn_kernel(jets_ref, w_emb_ref, b_emb_ref,
                 w_mp0_ref, b_mp0_ref, w_mp1_ref, b_mp1_ref,
                 w_mp2_ref, b_mp2_ref,
                 w_r1_ref, b_r1_ref, w_r2_ref, b_r2_ref,
                 out_ref, a_ref):
    w_emb = w_emb_ref[...]
    b_emb = b_emb_ref[...]
    mp_params = [(w_mp0_ref[...], b_mp0_ref[...]),
                 (w_mp1_ref[...], b_mp1_ref[...]),
                 (w_mp2_ref[...], b_mp2_ref[...])]
    w_r1 = w_r1_ref[...]
    b_r1 = b_r1_ref[...]
    w_r2 = w_r2_ref[...]
    b_r2 = b_r2_ref[...]

    hs = [jnp.tanh(_mm(jets_ref[j], w_emb) + b_emb) for j in range(_J)]
    a_s = [None] * _J
    for w, b in mp_params:
        logits = [jax.lax.dot_general(
            h, h, (((1,), (1,)), ((), ())),
            preferred_element_type=jnp.float32) * _SCALE for h in hs]
        ms = [jnp.max(l, axis=-1, keepdims=True) for l in logits]
        ps = [jnp.exp(l - m) for l, m in zip(logits, ms)]
        a_s = [p / jnp.sum(p, axis=-1, keepdims=True) for p in ps]
        msgs = [_mm(a, h) for a, h in zip(a_s, hs)]
        hs = [jnp.tanh(_mm(h, w[:_HID]) + _mm(msg, w[_HID:]) + b)
              for h, msg in zip(hs, msgs)]
    rs = [jnp.tanh(_mm(h, w_r1) + b_r1) for h in hs]
    r2s = [_mm(r, w_r2) for r in rs]
    for j in range(_J):
        out_ref[j] = jnp.sum(r2s[j], axis=0, keepdims=True) + _N * b_r2
        a_ref[j] = a_s[j]


def kernel(jets, W_emb, b_emb, W_mp0, b_mp0, W_mp1, b_mp1, W_mp2, b_mp2,
           W_r1, b_r1, W_r2, b_r2):
    B, N, F_IN = jets.shape
    HID = W_emb.shape[1]

    def rep(shape):
        return pl.BlockSpec(shape, lambda b: (0,) * len(shape))

    b_emb2 = b_emb.reshape(1, HID)
    b_mp0_2 = b_mp0.reshape(1, HID)
    b_mp1_2 = b_mp1.reshape(1, HID)
    b_mp2_2 = b_mp2.reshape(1, HID)
    b_r1_2 = b_r1.reshape(1, HID)
    b_r2_2 = b_r2.reshape(1, HID)

    out, a = pl.pallas_call(
        _mpnn_kernel,
        grid=(B // _J,),
        in_specs=[
            pl.BlockSpec((_J, N, F_IN), lambda b: (b, 0, 0)),
            rep((F_IN, HID)), rep((1, HID)),
            rep((2 * HID, HID)), rep((1, HID)),
            rep((2 * HID, HID)), rep((1, HID)),
            rep((2 * HID, HID)), rep((1, HID)),
            rep((HID, HID)), rep((1, HID)),
            rep((HID, HID)), rep((1, HID)),
        ],
        out_specs=[
            pl.BlockSpec((_J, 1, HID), lambda b: (b, 0, 0)),
            pl.BlockSpec((_J, N, N), lambda b: (b, 0, 0)),
        ],
        out_shape=[
            jax.ShapeDtypeStruct((B, 1, HID), jnp.float32),
            jax.ShapeDtypeStruct((B, N, N), jnp.float32),
        ],
        compiler_params=pltpu.CompilerParams(
            dimension_semantics=("parallel",),
        ),
    )(jets, W_emb, b_emb2, W_mp0, b_mp0_2, W_mp1, b_mp1_2, W_mp2, b_mp2_2,
      W_r1, b_r1_2, W_r2, b_r2_2)
    return (out.reshape(B, HID), a)
